# baseline (speedup 1.0000x reference)
import functools

import jax
import jax.numpy as jnp
from jax import lax
from jax.experimental import pallas as pl
from jax.experimental.pallas import tpu as pltpu
from jax.experimental.pallas import tpu_sc as plsc

N = 10000
E = 320000
D = 128
ED = 16
NP = 10240
NC = 2
NS = 16
NW = NC * NS
EPT = E // NW
VPT = EPT // 16
B = 80
NB = EPT // B
NSL = NP // NS

_MESH = plsc.VectorSubcoreMesh(core_axis_name="c", subcore_axis_name="s")
_SC_PARAMS = pltpu.CompilerParams(needs_layout_passes=False,
                                  use_tc_tiling_on_sc=False)


def _worker(c, s):
    return c * NS + s


def _zeros16():
    return jnp.zeros((16,), jnp.float32)


def _bcast16(ref, i):
    return plsc.load_gather(ref, [jnp.full((16,), i, jnp.int32)])


@functools.partial(
    pl.kernel,
    out_type=(
        jax.ShapeDtypeStruct((E,), jnp.float32),
        jax.ShapeDtypeStruct((NC * NP,), jnp.float32),
    ),
    mesh=_MESH,
    compiler_params=_SC_PARAMS,
    scratch_types=[
        pltpu.VMEM((N,), jnp.float32),
        pltpu.VMEM((N,), jnp.float32),
        pltpu.VMEM((EPT,), jnp.int32),
        pltpu.VMEM((EPT,), jnp.int32),
        pltpu.VMEM((NB, B), jnp.int32),
        pltpu.VMEM((EPT,), jnp.float32),
        pltpu.VMEM((EPT,), jnp.float32),
        pltpu.VMEM((NSL,), jnp.float32),
        pltpu.VMEM_SHARED((NP,), jnp.float32),
        pltpu.SemaphoreType.DMA,
    ],
)
def _sc_pass1(src_h, src2d_h, dst_h, r_h, p_h, q_h, expa_h, s1_h,
              pv, qv, sv, dv, sv2d, rv, eav, zv, s1s, sem):
    c = lax.axis_index("c")
    s = lax.axis_index("s")
    wid = _worker(c, s)
    eb = wid * EPT

    pltpu.sync_copy(p_h, pv)
    pltpu.sync_copy(q_h, qv)
    pltpu.sync_copy(src_h.at[pl.ds(eb, EPT)], sv)
    pltpu.sync_copy(dst_h.at[pl.ds(eb, EPT)], dv)
    pltpu.sync_copy(src2d_h.at[wid], sv2d)
    pltpu.sync_copy(r_h.at[pl.ds(eb, EPT)], rv)

    for i in range(NSL // 16):
        zv[pl.ds(i * 16, 16)] = _zeros16()
    pltpu.sync_copy(zv, s1s.at[pl.ds(s * NSL, NSL)])
    plsc.subcore_barrier()

    def body(i, _):
        sl = pl.ds(i * 16, 16)
        s16 = sv[sl]
        d16 = dv[sl]
        a16 = rv[sl] + plsc.load_gather(pv, [s16]) + plsc.load_gather(qv, [d16])
        eav[sl] = jnp.exp(a16)
        return 0

    lax.fori_loop(0, VPT, body, 0)

    def scat(j, _):
        pltpu.async_copy(eav.at[pl.ds(j * B, B)], s1s.at[sv2d.at[j]], sem,
                         add=True)

        @pl.when(j >= 8)
        def _():
            pltpu.make_async_copy(eav.at[pl.ds(0, B)], s1s.at[sv2d.at[0]],
                                  sem).wait()
        return 0

    lax.fori_loop(0, NB, scat, 0)
    for _ in range(8):
        pltpu.make_async_copy(eav.at[pl.ds(0, B)], s1s.at[sv2d.at[0]],
                              sem).wait()
    plsc.subcore_barrier()

    pltpu.sync_copy(s1s.at[pl.ds(s * NSL, NSL)],
                    s1_h.at[pl.ds(c * NP + s * NSL, NSL)])
    pltpu.sync_copy(eav, expa_h.at[pl.ds(eb, EPT)])


@functools.partial(
    pl.kernel,
    out_type=(
        jax.ShapeDtypeStruct((E,), jnp.float32),
        jax.ShapeDtypeStruct((E,), jnp.float32),
        jax.ShapeDtypeStruct((NC * NP,), jnp.float32),
    ),
    mesh=_MESH,
    compiler_params=_SC_PARAMS,
    scratch_types=[
        pltpu.VMEM((N,), jnp.float32),
        pltpu.VMEM((NP,), jnp.float32),
        pltpu.VMEM((NP,), jnp.float32),
        pltpu.VMEM((EPT,), jnp.int32),
        pltpu.VMEM((NB, B), jnp.int32),
        pltpu.VMEM((EPT,), jnp.float32),
        pltpu.VMEM((EPT,), jnp.float32),
        pltpu.VMEM((EPT,), jnp.float32),
        pltpu.VMEM((EPT,), jnp.float32),
        pltpu.VMEM((NSL,), jnp.float32),
        pltpu.VMEM_SHARED((NP,), jnp.float32),
        pltpu.SemaphoreType.DMA,
    ],
)
def _sc_pass2(src_h, dst2d_h, expa_h, u_h, t_h, s1_h, gamma_h, exps_h, s2_h,
              tv, s1v, s1b, sv, dv2d, eav, uv, gv, esv, zv, s2s, sem):
    c = lax.axis_index("c")
    s = lax.axis_index("s")
    wid = _worker(c, s)
    eb = wid * EPT

    pltpu.sync_copy(s1_h.at[pl.ds(0, NP)], s1v)
    pltpu.sync_copy(s1_h.at[pl.ds(NP, NP)], s1b)

    def comb(i, _):
        sl = pl.ds(i * 16, 16)
        s1v[sl] = s1v[sl] + s1b[sl]
        return 0

    lax.fori_loop(0, NP // 16, comb, 0)

    pltpu.sync_copy(t_h, tv)
    pltpu.sync_copy(src_h.at[pl.ds(eb, EPT)], sv)
    pltpu.sync_copy(dst2d_h.at[wid], dv2d)
    pltpu.sync_copy(expa_h.at[pl.ds(eb, EPT)], eav)
    pltpu.sync_copy(u_h.at[pl.ds(eb, EPT)], uv)

    for i in range(NSL // 16):
        zv[pl.ds(i * 16, 16)] = _zeros16()
    pltpu.sync_copy(zv, s2s.at[pl.ds(s * NSL, NSL)])
    plsc.subcore_barrier()

    def body(i, _):
        sl = pl.ds(i * 16, 16)
        s16 = sv[sl]
        g16 = eav[sl] / plsc.load_gather(s1v, [s16])
        gv[sl] = g16
        sc16 = plsc.load_gather(tv, [s16]) + g16 * uv[sl]
        esv[sl] = jnp.exp(sc16)
        return 0

    lax.fori_loop(0, VPT, body, 0)

    def scat(j, _):
        pltpu.async_copy(esv.at[pl.ds(j * B, B)], s2s.at[dv2d.at[j]], sem,
                         add=True)

        @pl.when(j >= 8)
        def _():
            pltpu.make_async_copy(esv.at[pl.ds(0, B)], s2s.at[dv2d.at[0]],
                                  sem).wait()
        return 0

    lax.fori_loop(0, NB, scat, 0)
    for _ in range(8):
        pltpu.make_async_copy(esv.at[pl.ds(0, B)], s2s.at[dv2d.at[0]],
                              sem).wait()
    plsc.subcore_barrier()

    pltpu.sync_copy(s2s.at[pl.ds(s * NSL, NSL)],
                    s2_h.at[pl.ds(c * NP + s * NSL, NSL)])
    pltpu.sync_copy(gv, gamma_h.at[pl.ds(eb, EPT)])
    pltpu.sync_copy(esv, exps_h.at[pl.ds(eb, EPT)])


EPT3 = E // NS
NB3 = EPT3 // B
DH = D // 2
DC = DH + ED
NBS = 50
NSC = NB3 // NBS
NCMB = 2048


def _bcast2d(ref, row, col):
    return plsc.load_gather(ref, [jnp.full((16,), row, jnp.int32),
                                  jnp.full((16,), col, jnp.int32)])


@functools.partial(
    pl.kernel,
    out_type=jax.ShapeDtypeStruct((NC, NP, DC), jnp.float32),
    mesh=_MESH,
    compiler_params=_SC_PARAMS,
    scratch_types=[
        pltpu.VMEM((NP,), jnp.float32),
        pltpu.VMEM((NCMB,), jnp.float32),
        pltpu.VMEM((NBS, B), jnp.int32),
        pltpu.VMEM((NBS, B), jnp.int32),
        pltpu.VMEM((NBS, B), jnp.float32),
        pltpu.VMEM((NBS, B), jnp.float32),
        pltpu.VMEM((B, DH), jnp.float32),
        pltpu.VMEM((B, DH), jnp.float32),
        pltpu.VMEM((B, DH), jnp.float32),
        pltpu.VMEM((ED, B), jnp.float32),
        pltpu.VMEM((ED, B), jnp.float32),
        pltpu.VMEM((ED, B), jnp.float32),
        pltpu.VMEM((B, DC), jnp.float32),
        pltpu.VMEM((B, DC), jnp.float32),
        pltpu.VMEM((B, DC), jnp.float32),
        pltpu.SemaphoreType.DMA,
        pltpu.SemaphoreType.DMA,
        pltpu.SemaphoreType.DMA,
        pltpu.SemaphoreType.DMA,
        pltpu.SemaphoreType.DMA,
        pltpu.SemaphoreType.DMA,
        pltpu.SemaphoreType.DMA,
        pltpu.SemaphoreType.DMA,
        pltpu.SemaphoreType.DMA,
        pltpu.VMEM_SHARED((NP, DC), jnp.float32),
    ],
)
def _sc_pass3(src3_h, dst3_h, exps3_h, gamma3_h, s2_h, xs_h, eft_h,
              acc_h,
              s2v, ctmp, svx, dvx, esx, gvx, gbuf0, gbuf1, gbuf2,
              efg0, efg1, efg2, sbuf0, sbuf1, sbuf2,
              sem_g0, sem_g1, sem_g2, sem_e0, sem_e1, sem_e2,
              sem_s0, sem_s1, sem_s2, accs):
    c = lax.axis_index("c")
    s = lax.axis_index("s")
    eb = s * EPT3
    gbufs = (gbuf0, gbuf1, gbuf2)
    efgs = (efg0, efg1, efg2)
    sbufs = (sbuf0, sbuf1, sbuf2)
    sem_gs = (sem_g0, sem_g1, sem_g2)
    sem_es = (sem_e0, sem_e1, sem_e2)
    sem_ss = (sem_s0, sem_s1, sem_s2)

    def comb(m, _):
        sl = pl.ds(m * NCMB, NCMB)
        pltpu.sync_copy(s2_h.at[sl], s2v.at[sl])
        pltpu.sync_copy(s2_h.at[pl.ds(NP + m * NCMB, NCMB)], ctmp)

        def add16(i, _):
            sl16 = pl.ds(m * NCMB + i * 16, 16)
            s2v[sl16] = s2v[sl16] + ctmp[pl.ds(i * 16, 16)]
            return 0

        lax.fori_loop(0, NCMB // 16, add16, 0)
        return 0

    lax.fori_loop(0, NP // NCMB, comb, 0)

    def zrow(e, _):
        for k in range(DC // 16):
            sbuf0[e, pl.ds(k * 16, 16)] = _zeros16()
        return 0

    lax.fori_loop(0, B, zrow, 0)
    for k in range(NSL // B):
        pltpu.sync_copy(sbuf0, accs.at[pl.ds(s * NSL + k * B, B)])
    plsc.subcore_barrier()

    def superchunk(sc, _):
        rsl = pl.ds(sc * NBS, NBS)
        pltpu.sync_copy(src3_h.at[s, rsl], svx)
        pltpu.sync_copy(dst3_h.at[s, rsl], dvx)
        pltpu.sync_copy(exps3_h.at[s, rsl], esx)
        pltpu.sync_copy(gamma3_h.at[s, rsl], gvx)

        def arow(j, _):
            for k in range(B // 16):
                sl = pl.ds(k * 16, 16)
                a16 = esx[j, sl] / plsc.load_gather(s2v, [dvx[j, sl]])
                esx[j, sl] = a16
                gvx[j, sl] = (a16 * 0.5) * gvx[j, sl]
            return 0

        lax.fori_loop(0, NBS, arow, 0)

        csc = sc * NBS
        iota = lax.iota(jnp.int32, 16)

        def issue_in(j, p):
            pltpu.async_copy(xs_h.at[c].at[svx.at[j]], gbufs[p], sem_gs[p])
            pltpu.async_copy(eft_h.at[:, pl.ds(eb + (csc + j) * B, B)],
                             efgs[p], sem_es[p])

        def wait_x(p):
            pltpu.make_async_copy(xs_h.at[c].at[svx.at[0]], gbufs[p],
                                  sem_gs[p]).wait()

        def wait_ef(p):
            pltpu.make_async_copy(eft_h.at[:, pl.ds(eb, B)], efgs[p],
                                  sem_es[p]).wait()

        def wait_out(p):
            pltpu.make_async_copy(sbufs[p], accs.at[dvx.at[0]],
                                  sem_ss[p]).wait()

        def slot(j, p, issue, wait_prev_out):
            if issue:
                issue_in(j + 2, (p + 2) % 3)
            if wait_prev_out:
                wait_out(p)

            wait_ef(p)
            jcol = jnp.full((16,), j, jnp.int32)

            def scale_e(m, _):
                sl = pl.ds(m * 16, 16)
                v16 = gvx[j, sl]
                rows16 = iota + m * 16
                for k in range(ED):
                    plsc.store_scatter(
                        sbufs[p], [rows16, jnp.full((16,), DH + k, jnp.int32)],
                        efgs[p][k, sl] * v16)
                return 0

            lax.fori_loop(0, B // 16, scale_e, 0)

            wait_x(p)

            def scale(e2, _):
                for e in (2 * e2, 2 * e2 + 1):
                    ab = plsc.load_gather(
                        esx, [jcol, jnp.full((16,), e, jnp.int32)])
                    for k in range(DH // 16):
                        sl = pl.ds(k * 16, 16)
                        sbufs[p][e, sl] = gbufs[p][e, sl] * ab
                return 0

            lax.fori_loop(0, B // 2, scale, 0)
            pltpu.async_copy(sbufs[p], accs.at[dvx.at[j]], sem_ss[p],
                             add=True)

        def triple(jj, _):
            j = 3 * jj
            slot(j, 0, True, True)
            slot(j + 1, 1, True, True)
            slot(j + 2, 2, True, True)
            return 0

        issue_in(0, 0)
        issue_in(1, 1)
        slot(0, 0, True, False)
        slot(1, 1, True, False)
        slot(2, 2, True, False)
        lax.fori_loop(1, 15, triple, 0)
        slot(45, 0, True, True)
        slot(46, 1, True, True)
        slot(47, 2, True, True)
        slot(48, 0, False, True)
        slot(49, 1, False, True)
        wait_out(2)
        wait_out(0)
        wait_out(1)
        return 0

    lax.fori_loop(0, NSC, superchunk, 0)
    plsc.subcore_barrier()

    pltpu.sync_copy(accs.at[pl.ds(s * NSL, NSL)],
                    acc_h.at[c, pl.ds(s * NSL, NSL)])


def _proj_body(x_ref, w_ref, o_ref):
    o_ref[...] = jnp.dot(x_ref[...], w_ref[...],
                         preferred_element_type=jnp.float32)


def _node_proj(x, w3):
    blk = 1000
    return pl.pallas_call(
        _proj_body,
        grid=(N // blk,),
        in_specs=[
            pl.BlockSpec((blk, D), lambda i: (i, 0)),
            pl.BlockSpec((D, 8), lambda i: (0, 0)),
        ],
        out_specs=pl.BlockSpec((blk, 8), lambda i: (i, 0)),
        out_shape=jax.ShapeDtypeStruct((N, 8), jnp.float32),
    )(x, w3)


def _edge_proj_body(eft_ref, we_ref, w2_ref, r_ref, u_ref):
    blk = eft_ref[...]
    r_ref[...] = jnp.sum(blk * we_ref[...], axis=0)
    u_ref[...] = jnp.sum(blk * w2_ref[...], axis=0)


def _edge_proj(eft, we_col, w2_col):
    return pl.pallas_call(
        _edge_proj_body,
        grid=(1,),
        in_specs=[
            pl.BlockSpec((ED, E), lambda i: (0, 0)),
            pl.BlockSpec((ED, 1), lambda i: (0, 0)),
            pl.BlockSpec((ED, 1), lambda i: (0, 0)),
        ],
        out_specs=[
            pl.BlockSpec((E,), lambda i: (0,)),
            pl.BlockSpec((E,), lambda i: (0,)),
        ],
        out_shape=[
            jax.ShapeDtypeStruct((E,), jnp.float32),
            jax.ShapeDtypeStruct((E,), jnp.float32),
        ],
    )(eft, we_col, w2_col)


def _final_body(x_ref, a0_ref, a1_ref, wn_ref, wc0_ref, wc1_ref, o_ref):
    acc = jnp.dot(x_ref[...], wn_ref[...], preferred_element_type=jnp.float32)
    acc = acc + jnp.dot(a0_ref[...], wc0_ref[...],
                        preferred_element_type=jnp.float32)
    acc = acc + jnp.dot(a1_ref[...], wc1_ref[...],
                        preferred_element_type=jnp.float32)
    o_ref[...] = acc


def _final_linear(x, a0, a1, wn_t, wc0, wc1):
    blk = 1000
    return pl.pallas_call(
        _final_body,
        grid=(N // blk,),
        in_specs=[
            pl.BlockSpec((blk, D), lambda i: (i, 0)),
            pl.BlockSpec((blk, DC), lambda i: (i, 0)),
            pl.BlockSpec((blk, DC), lambda i: (i, 0)),
            pl.BlockSpec((D, D), lambda i: (0, 0)),
            pl.BlockSpec((DC, D), lambda i: (0, 0)),
            pl.BlockSpec((DC, D), lambda i: (0, 0)),
        ],
        out_specs=pl.BlockSpec((blk, D), lambda i: (i, 0)),
        out_shape=jax.ShapeDtypeStruct((N, D), jnp.float32),
    )(x, a0, a1, wn_t, wc0, wc1)


def kernel(node_features, edge_features, edge_index, W_group_attn,
           W_src_attn, W_linear):
    x = node_features
    eft = edge_features.T
    src = edge_index[0]
    dst = edge_index[1]
    src2d, dst2d, src2d3, dst2d3 = lax.optimization_barrier(
        (src.reshape(NW, NB, B), dst.reshape(NW, NB, B),
         src.reshape(NS, NB3, B), dst.reshape(NS, NB3, B)))
    xs = jnp.stack([x[:, :DH], x[:, DH:]])

    w3 = jnp.zeros((D, 8), jnp.float32)
    w3 = w3.at[:, 0].set(W_group_attn[0, ED:ED + D])
    w3 = w3.at[:, 1].set(W_group_attn[0, ED + D:ED + 2 * D])
    w3 = w3.at[:, 2].set(W_src_attn[0, :D])
    we_col = W_group_attn[0, :ED].reshape(ED, 1)
    w2_col = W_src_attn[0, D:D + ED].reshape(ED, 1)

    pqt = _node_proj(x, w3)
    r, u = _edge_proj(eft, we_col, w2_col)
    p = pqt[:, 0]
    q = pqt[:, 1]
    t = pqt[:, 2]

    expa, s1 = _sc_pass1(src, src2d, dst, r, p, q)
    gamma, exps, s2 = _sc_pass2(src, dst2d, expa, u, t, s1)
    exps3, gamma3 = lax.optimization_barrier(
        (exps.reshape(NS, NB3, B), gamma.reshape(NS, NB3, B)))
    acc = _sc_pass3(src2d3, dst2d3, exps3, gamma3, s2, xs, eft)

    wn_t = W_linear[:, :D].T
    wc0 = jnp.concatenate([W_linear[:, D:D + DH].T,
                           W_linear[:, 2 * D:2 * D + ED].T], axis=0)
    wc1 = jnp.concatenate([W_linear[:, D + DH:2 * D].T,
                           W_linear[:, 2 * D:2 * D + ED].T], axis=0)
    return _final_linear(x, acc[0, :N], acc[1, :N], wn_t, wc0, wc1)


# speedup vs baseline: 1.0123x; 1.0123x over previous
import functools

import jax
import jax.numpy as jnp
from jax import lax
from jax.experimental import pallas as pl
from jax.experimental.pallas import tpu as pltpu
from jax.experimental.pallas import tpu_sc as plsc

N = 10000
E = 320000
D = 128
ED = 16
NP = 10240
NC = 2
NS = 16
NW = NC * NS
EPT = E // NW
VPT = EPT // 16
B = 80
NB = EPT // B
NSL = NP // NS

_MESH = plsc.VectorSubcoreMesh(core_axis_name="c", subcore_axis_name="s")
_SC_PARAMS = pltpu.CompilerParams(needs_layout_passes=False,
                                  use_tc_tiling_on_sc=False)


def _worker(c, s):
    return c * NS + s


def _zeros16():
    return jnp.zeros((16,), jnp.float32)


def _bcast16(ref, i):
    return plsc.load_gather(ref, [jnp.full((16,), i, jnp.int32)])


@functools.partial(
    pl.kernel,
    out_type=(
        jax.ShapeDtypeStruct((E,), jnp.float32),
        jax.ShapeDtypeStruct((NC * NP,), jnp.float32),
    ),
    mesh=_MESH,
    compiler_params=_SC_PARAMS,
    scratch_types=[
        pltpu.VMEM((N,), jnp.float32),
        pltpu.VMEM((N,), jnp.float32),
        pltpu.VMEM((EPT,), jnp.int32),
        pltpu.VMEM((EPT,), jnp.int32),
        pltpu.VMEM((NB, B), jnp.int32),
        pltpu.VMEM((EPT,), jnp.float32),
        pltpu.VMEM((EPT,), jnp.float32),
        pltpu.VMEM((NSL,), jnp.float32),
        pltpu.VMEM_SHARED((NP,), jnp.float32),
        pltpu.SemaphoreType.DMA,
    ],
)
def _sc_pass1(src_h, src2d_h, dst_h, r_h, p_h, q_h, expa_h, s1_h,
              pv, qv, sv, dv, sv2d, rv, eav, zv, s1s, sem):
    c = lax.axis_index("c")
    s = lax.axis_index("s")
    wid = _worker(c, s)
    eb = wid * EPT

    pltpu.sync_copy(p_h, pv)
    pltpu.sync_copy(q_h, qv)
    pltpu.sync_copy(src_h.at[pl.ds(eb, EPT)], sv)
    pltpu.sync_copy(dst_h.at[pl.ds(eb, EPT)], dv)
    pltpu.sync_copy(src2d_h.at[wid], sv2d)
    pltpu.sync_copy(r_h.at[pl.ds(eb, EPT)], rv)

    for i in range(NSL // 16):
        zv[pl.ds(i * 16, 16)] = _zeros16()
    pltpu.sync_copy(zv, s1s.at[pl.ds(s * NSL, NSL)])
    plsc.subcore_barrier()

    def body(i, _):
        sl = pl.ds(i * 16, 16)
        s16 = sv[sl]
        d16 = dv[sl]
        a16 = rv[sl] + plsc.load_gather(pv, [s16]) + plsc.load_gather(qv, [d16])
        eav[sl] = jnp.exp(a16)
        return 0

    lax.fori_loop(0, VPT, body, 0)

    def scat(j, _):
        pltpu.async_copy(eav.at[pl.ds(j * B, B)], s1s.at[sv2d.at[j]], sem,
                         add=True)

        @pl.when(j >= 8)
        def _():
            pltpu.make_async_copy(eav.at[pl.ds(0, B)], s1s.at[sv2d.at[0]],
                                  sem).wait()
        return 0

    lax.fori_loop(0, NB, scat, 0)
    for _ in range(8):
        pltpu.make_async_copy(eav.at[pl.ds(0, B)], s1s.at[sv2d.at[0]],
                              sem).wait()
    plsc.subcore_barrier()

    pltpu.sync_copy(s1s.at[pl.ds(s * NSL, NSL)],
                    s1_h.at[pl.ds(c * NP + s * NSL, NSL)])
    pltpu.sync_copy(eav, expa_h.at[pl.ds(eb, EPT)])


@functools.partial(
    pl.kernel,
    out_type=(
        jax.ShapeDtypeStruct((E,), jnp.float32),
        jax.ShapeDtypeStruct((E,), jnp.float32),
        jax.ShapeDtypeStruct((NC * NP,), jnp.float32),
    ),
    mesh=_MESH,
    compiler_params=_SC_PARAMS,
    scratch_types=[
        pltpu.VMEM((N,), jnp.float32),
        pltpu.VMEM((NP,), jnp.float32),
        pltpu.VMEM((NP,), jnp.float32),
        pltpu.VMEM((EPT,), jnp.int32),
        pltpu.VMEM((NB, B), jnp.int32),
        pltpu.VMEM((EPT,), jnp.float32),
        pltpu.VMEM((EPT,), jnp.float32),
        pltpu.VMEM((EPT,), jnp.float32),
        pltpu.VMEM((EPT,), jnp.float32),
        pltpu.VMEM((NSL,), jnp.float32),
        pltpu.VMEM_SHARED((NP,), jnp.float32),
        pltpu.SemaphoreType.DMA,
    ],
)
def _sc_pass2(src_h, dst2d_h, expa_h, u_h, t_h, s1_h, gamma_h, exps_h, s2_h,
              tv, s1v, s1b, sv, dv2d, eav, uv, gv, esv, zv, s2s, sem):
    c = lax.axis_index("c")
    s = lax.axis_index("s")
    wid = _worker(c, s)
    eb = wid * EPT

    pltpu.sync_copy(s1_h.at[pl.ds(0, NP)], s1v)
    pltpu.sync_copy(s1_h.at[pl.ds(NP, NP)], s1b)

    def comb(i, _):
        sl = pl.ds(i * 16, 16)
        s1v[sl] = s1v[sl] + s1b[sl]
        return 0

    lax.fori_loop(0, NP // 16, comb, 0)

    pltpu.sync_copy(t_h, tv)
    pltpu.sync_copy(src_h.at[pl.ds(eb, EPT)], sv)
    pltpu.sync_copy(dst2d_h.at[wid], dv2d)
    pltpu.sync_copy(expa_h.at[pl.ds(eb, EPT)], eav)
    pltpu.sync_copy(u_h.at[pl.ds(eb, EPT)], uv)

    for i in range(NSL // 16):
        zv[pl.ds(i * 16, 16)] = _zeros16()
    pltpu.sync_copy(zv, s2s.at[pl.ds(s * NSL, NSL)])
    plsc.subcore_barrier()

    def body(i, _):
        sl = pl.ds(i * 16, 16)
        s16 = sv[sl]
        g16 = eav[sl] / plsc.load_gather(s1v, [s16])
        gv[sl] = g16
        sc16 = plsc.load_gather(tv, [s16]) + g16 * uv[sl]
        esv[sl] = jnp.exp(sc16)
        return 0

    lax.fori_loop(0, VPT, body, 0)

    def scat(j, _):
        pltpu.async_copy(esv.at[pl.ds(j * B, B)], s2s.at[dv2d.at[j]], sem,
                         add=True)

        @pl.when(j >= 8)
        def _():
            pltpu.make_async_copy(esv.at[pl.ds(0, B)], s2s.at[dv2d.at[0]],
                                  sem).wait()
        return 0

    lax.fori_loop(0, NB, scat, 0)
    for _ in range(8):
        pltpu.make_async_copy(esv.at[pl.ds(0, B)], s2s.at[dv2d.at[0]],
                              sem).wait()
    plsc.subcore_barrier()

    pltpu.sync_copy(s2s.at[pl.ds(s * NSL, NSL)],
                    s2_h.at[pl.ds(c * NP + s * NSL, NSL)])
    pltpu.sync_copy(gv, gamma_h.at[pl.ds(eb, EPT)])
    pltpu.sync_copy(esv, exps_h.at[pl.ds(eb, EPT)])


EPT3 = E // NS
NB3 = EPT3 // B
DH = D // 2
DC = DH + ED
NBS = 50
NSC = NB3 // NBS
NCMB = 2048


def _bcast2d(ref, row, col):
    return plsc.load_gather(ref, [jnp.full((16,), row, jnp.int32),
                                  jnp.full((16,), col, jnp.int32)])


@functools.partial(
    pl.kernel,
    out_type=jax.ShapeDtypeStruct((NC, NP, DC), jnp.float32),
    mesh=_MESH,
    compiler_params=_SC_PARAMS,
    scratch_types=[
        pltpu.VMEM((NP,), jnp.float32),
        pltpu.VMEM((NCMB,), jnp.float32),
        pltpu.VMEM((NBS, B), jnp.int32),
        pltpu.VMEM((NBS, B), jnp.int32),
        pltpu.VMEM((NBS, B), jnp.float32),
        pltpu.VMEM((NBS, B), jnp.float32),
        pltpu.VMEM((B, DH), jnp.float32),
        pltpu.VMEM((B, DH), jnp.float32),
        pltpu.VMEM((ED, B), jnp.float32),
        pltpu.VMEM((ED, B), jnp.float32),
        pltpu.VMEM((B, DC), jnp.float32),
        pltpu.VMEM((B, DC), jnp.float32),
        pltpu.SemaphoreType.DMA,
        pltpu.SemaphoreType.DMA,
        pltpu.SemaphoreType.DMA,
        pltpu.SemaphoreType.DMA,
        pltpu.SemaphoreType.DMA,
        pltpu.SemaphoreType.DMA,
        pltpu.VMEM_SHARED((NP, DC), jnp.float32),
    ],
)
def _sc_pass3(src3_h, dst3_h, exps3_h, gamma3_h, s2_h, x0_h, x1_h, eft_h,
              acc_h,
              s2v, ctmp, svx, dvx, esx, gvx, gbuf0, gbuf1, efg0, efg1,
              sbuf0, sbuf1, sem_g0, sem_g1, sem_e0, sem_e1, sem_s0,
              sem_s1, accs):
    c = lax.axis_index("c")
    s = lax.axis_index("s")
    eb = s * EPT3
    gbufs = (gbuf0, gbuf1)
    efgs = (efg0, efg1)
    sbufs = (sbuf0, sbuf1)
    sem_gs = (sem_g0, sem_g1)
    sem_es = (sem_e0, sem_e1)
    sem_ss = (sem_s0, sem_s1)

    def comb(m, _):
        sl = pl.ds(m * NCMB, NCMB)
        pltpu.sync_copy(s2_h.at[sl], s2v.at[sl])
        pltpu.sync_copy(s2_h.at[pl.ds(NP + m * NCMB, NCMB)], ctmp)

        def add16(i, _):
            sl16 = pl.ds(m * NCMB + i * 16, 16)
            s2v[sl16] = s2v[sl16] + ctmp[pl.ds(i * 16, 16)]
            return 0

        lax.fori_loop(0, NCMB // 16, add16, 0)
        return 0

    lax.fori_loop(0, NP // NCMB, comb, 0)

    def zrow(e, _):
        for k in range(DC // 16):
            sbuf0[e, pl.ds(k * 16, 16)] = _zeros16()
        return 0

    lax.fori_loop(0, B, zrow, 0)
    for k in range(NSL // B):
        pltpu.sync_copy(sbuf0, accs.at[pl.ds(s * NSL + k * B, B)])
    plsc.subcore_barrier()

    def issue_in(j, gj, p):
        @pl.when(c == 0)
        def _():
            pltpu.async_copy(x0_h.at[svx.at[j]], gbufs[p], sem_gs[p])

        @pl.when(c == 1)
        def _():
            pltpu.async_copy(x1_h.at[svx.at[j]], gbufs[p], sem_gs[p])

        pltpu.async_copy(eft_h.at[:, pl.ds(eb + gj * B, B)], efgs[p],
                         sem_es[p])

    def wait_x(p):
        pltpu.make_async_copy(x0_h.at[svx.at[0]], gbufs[p], sem_gs[p]).wait()

    def wait_ef(p):
        pltpu.make_async_copy(eft_h.at[:, pl.ds(eb, B)], efgs[p],
                              sem_es[p]).wait()

    def wait_out(p):
        pltpu.make_async_copy(sbufs[p], accs.at[dvx.at[0]], sem_ss[p]).wait()

    def superchunk(sc, _):
        rsl = pl.ds(sc * NBS, NBS)
        pltpu.sync_copy(src3_h.at[s, rsl], svx)
        pltpu.sync_copy(dst3_h.at[s, rsl], dvx)
        pltpu.sync_copy(exps3_h.at[s, rsl], esx)
        pltpu.sync_copy(gamma3_h.at[s, rsl], gvx)

        def arow(j, _):
            for k in range(B // 16):
                sl = pl.ds(k * 16, 16)
                a16 = esx[j, sl] / plsc.load_gather(s2v, [dvx[j, sl]])
                esx[j, sl] = a16
                gvx[j, sl] = (a16 * 0.5) * gvx[j, sl]
            return 0

        lax.fori_loop(0, NBS, arow, 0)

        csc = sc * NBS

        iota = lax.iota(jnp.int32, 16)

        def slot(j, p):
            @pl.when(j + 1 < NBS)
            def _():
                issue_in(j + 1, csc + j + 1, 1 - p)

            @pl.when(j >= 2)
            def _():
                wait_out(p)

            wait_ef(p)
            jcol = jnp.full((16,), j, jnp.int32)

            def scale_e(m, _):
                sl = pl.ds(m * 16, 16)
                v16 = gvx[j, sl]
                rows16 = iota + m * 16
                for k in range(ED):
                    plsc.store_scatter(
                        sbufs[p], [rows16, jnp.full((16,), DH + k, jnp.int32)],
                        efgs[p][k, sl] * v16)
                return 0

            lax.fori_loop(0, B // 16, scale_e, 0)

            wait_x(p)

            def scale(e2, _):
                for e in (2 * e2, 2 * e2 + 1):
                    ab = plsc.load_gather(
                        esx, [jcol, jnp.full((16,), e, jnp.int32)])
                    for k in range(DH // 16):
                        sl = pl.ds(k * 16, 16)
                        sbufs[p][e, sl] = gbufs[p][e, sl] * ab
                return 0

            lax.fori_loop(0, B // 2, scale, 0)
            pltpu.async_copy(sbufs[p], accs.at[dvx.at[j]], sem_ss[p],
                             add=True)
            return 0

        def pair(jj, _):
            slot(2 * jj, 0)
            slot(2 * jj + 1, 1)
            return 0

        issue_in(0, csc, 0)
        lax.fori_loop(0, NBS // 2, pair, 0)
        wait_out(0)
        wait_out(1)
        return 0

    lax.fori_loop(0, NSC, superchunk, 0)
    plsc.subcore_barrier()

    pltpu.sync_copy(accs.at[pl.ds(s * NSL, NSL)],
                    acc_h.at[c, pl.ds(s * NSL, NSL)])


def _proj_body(x_ref, w_ref, o_ref):
    o_ref[...] = jnp.dot(x_ref[...], w_ref[...],
                         preferred_element_type=jnp.float32)


def _node_proj(x, w3):
    blk = 1000
    return pl.pallas_call(
        _proj_body,
        grid=(N // blk,),
        in_specs=[
            pl.BlockSpec((blk, D), lambda i: (i, 0)),
            pl.BlockSpec((D, 8), lambda i: (0, 0)),
        ],
        out_specs=pl.BlockSpec((blk, 8), lambda i: (i, 0)),
        out_shape=jax.ShapeDtypeStruct((N, 8), jnp.float32),
    )(x, w3)


def _edge_proj_body(eft_ref, we_ref, w2_ref, r_ref, u_ref):
    blk = eft_ref[...]
    r_ref[...] = jnp.sum(blk * we_ref[...], axis=0)
    u_ref[...] = jnp.sum(blk * w2_ref[...], axis=0)


def _edge_proj(eft, we_col, w2_col):
    return pl.pallas_call(
        _edge_proj_body,
        grid=(1,),
        in_specs=[
            pl.BlockSpec((ED, E), lambda i: (0, 0)),
            pl.BlockSpec((ED, 1), lambda i: (0, 0)),
            pl.BlockSpec((ED, 1), lambda i: (0, 0)),
        ],
        out_specs=[
            pl.BlockSpec((E,), lambda i: (0,)),
            pl.BlockSpec((E,), lambda i: (0,)),
        ],
        out_shape=[
            jax.ShapeDtypeStruct((E,), jnp.float32),
            jax.ShapeDtypeStruct((E,), jnp.float32),
        ],
    )(eft, we_col, w2_col)


def _final_body(x_ref, a0_ref, a1_ref, wn_ref, wc0_ref, wc1_ref, o_ref):
    acc = jnp.dot(x_ref[...], wn_ref[...], preferred_element_type=jnp.float32)
    acc = acc + jnp.dot(a0_ref[...], wc0_ref[...],
                        preferred_element_type=jnp.float32)
    acc = acc + jnp.dot(a1_ref[...], wc1_ref[...],
                        preferred_element_type=jnp.float32)
    o_ref[...] = acc


def _final_linear(x, a0, a1, wn_t, wc0, wc1):
    blk = 1000
    return pl.pallas_call(
        _final_body,
        grid=(N // blk,),
        in_specs=[
            pl.BlockSpec((blk, D), lambda i: (i, 0)),
            pl.BlockSpec((blk, DC), lambda i: (i, 0)),
            pl.BlockSpec((blk, DC), lambda i: (i, 0)),
            pl.BlockSpec((D, D), lambda i: (0, 0)),
            pl.BlockSpec((DC, D), lambda i: (0, 0)),
            pl.BlockSpec((DC, D), lambda i: (0, 0)),
        ],
        out_specs=pl.BlockSpec((blk, D), lambda i: (i, 0)),
        out_shape=jax.ShapeDtypeStruct((N, D), jnp.float32),
    )(x, a0, a1, wn_t, wc0, wc1)


def kernel(node_features, edge_features, edge_index, W_group_attn,
           W_src_attn, W_linear):
    x = node_features
    eft = edge_features.T
    src = edge_index[0]
    dst = edge_index[1]
    src2d, dst2d, src2d3, dst2d3 = lax.optimization_barrier(
        (src.reshape(NW, NB, B), dst.reshape(NW, NB, B),
         src.reshape(NS, NB3, B), dst.reshape(NS, NB3, B)))
    x0 = x[:, :DH]
    x1 = x[:, DH:]

    w3 = jnp.zeros((D, 8), jnp.float32)
    w3 = w3.at[:, 0].set(W_group_attn[0, ED:ED + D])
    w3 = w3.at[:, 1].set(W_group_attn[0, ED + D:ED + 2 * D])
    w3 = w3.at[:, 2].set(W_src_attn[0, :D])
    we_col = W_group_attn[0, :ED].reshape(ED, 1)
    w2_col = W_src_attn[0, D:D + ED].reshape(ED, 1)

    pqt = _node_proj(x, w3)
    r, u = _edge_proj(eft, we_col, w2_col)
    p = pqt[:, 0]
    q = pqt[:, 1]
    t = pqt[:, 2]

    expa, s1 = _sc_pass1(src, src2d, dst, r, p, q)
    gamma, exps, s2 = _sc_pass2(src, dst2d, expa, u, t, s1)
    exps3, gamma3 = lax.optimization_barrier(
        (exps.reshape(NS, NB3, B), gamma.reshape(NS, NB3, B)))
    acc = _sc_pass3(src2d3, dst2d3, exps3, gamma3, s2, x0, x1, eft)

    wn_t = W_linear[:, :D].T
    wc0 = jnp.concatenate([W_linear[:, D:D + DH].T,
                           W_linear[:, 2 * D:2 * D + ED].T], axis=0)
    wc1 = jnp.concatenate([W_linear[:, D + DH:2 * D].T,
                           W_linear[:, 2 * D:2 * D + ED].T], axis=0)
    return _final_linear(x, acc[0, :N], acc[1, :N], wn_t, wc0, wc1)


# speedup vs baseline: 1.0213x; 1.0089x over previous
import functools

import jax
import jax.numpy as jnp
from jax import lax
from jax.experimental import pallas as pl
from jax.experimental.pallas import tpu as pltpu
from jax.experimental.pallas import tpu_sc as plsc

N = 10000
E = 320000
D = 128
ED = 16
NP = 10240
NC = 2
NS = 16
NW = NC * NS
EPT = E // NW
VPT = EPT // 16
B = 80
NB = EPT // B
NSL = NP // NS
EPT3 = E // NS
NB3 = EPT3 // B

_MESH = plsc.VectorSubcoreMesh(core_axis_name="c", subcore_axis_name="s")
_SC_PARAMS = pltpu.CompilerParams(needs_layout_passes=False,
                                  use_tc_tiling_on_sc=False)


def _worker(c, s):
    return c * NS + s


def _zeros16():
    return jnp.zeros((16,), jnp.float32)


def _bcast16(ref, i):
    return plsc.load_gather(ref, [jnp.full((16,), i, jnp.int32)])


@functools.partial(
    pl.kernel,
    out_type=(
        jax.ShapeDtypeStruct((E,), jnp.float32),
        jax.ShapeDtypeStruct((NC * NP,), jnp.float32),
    ),
    mesh=_MESH,
    compiler_params=_SC_PARAMS,
    scratch_types=[
        pltpu.VMEM((N,), jnp.float32),
        pltpu.VMEM((N,), jnp.float32),
        pltpu.VMEM((EPT,), jnp.int32),
        pltpu.VMEM((EPT,), jnp.int32),
        pltpu.VMEM((NB, B), jnp.int32),
        pltpu.VMEM((EPT,), jnp.float32),
        pltpu.VMEM((EPT,), jnp.float32),
        pltpu.VMEM((NSL,), jnp.float32),
        pltpu.VMEM_SHARED((NP,), jnp.float32),
        pltpu.SemaphoreType.DMA,
    ],
)
def _sc_pass1(src_h, src2d_h, dst_h, r_h, p_h, q_h, expa_h, s1_h,
              pv, qv, sv, dv, sv2d, rv, eav, zv, s1s, sem):
    c = lax.axis_index("c")
    s = lax.axis_index("s")
    wid = _worker(c, s)
    eb = wid * EPT

    pltpu.sync_copy(p_h, pv)
    pltpu.sync_copy(q_h, qv)
    pltpu.sync_copy(src_h.at[pl.ds(eb, EPT)], sv)
    pltpu.sync_copy(dst_h.at[pl.ds(eb, EPT)], dv)
    pltpu.sync_copy(src2d_h.at[wid], sv2d)
    pltpu.sync_copy(r_h.at[pl.ds(eb, EPT)], rv)

    for i in range(NSL // 16):
        zv[pl.ds(i * 16, 16)] = _zeros16()
    pltpu.sync_copy(zv, s1s.at[pl.ds(s * NSL, NSL)])
    plsc.subcore_barrier()

    def body(i, _):
        sl = pl.ds(i * 16, 16)
        s16 = sv[sl]
        d16 = dv[sl]
        a16 = rv[sl] + plsc.load_gather(pv, [s16]) + plsc.load_gather(qv, [d16])
        eav[sl] = jnp.exp(a16)
        return 0

    lax.fori_loop(0, VPT, body, 0)

    def scat(j, _):
        pltpu.async_copy(eav.at[pl.ds(j * B, B)], s1s.at[sv2d.at[j]], sem,
                         add=True)

        @pl.when(j >= 8)
        def _():
            pltpu.make_async_copy(eav.at[pl.ds(0, B)], s1s.at[sv2d.at[0]],
                                  sem).wait()
        return 0

    lax.fori_loop(0, NB, scat, 0)
    for _ in range(8):
        pltpu.make_async_copy(eav.at[pl.ds(0, B)], s1s.at[sv2d.at[0]],
                              sem).wait()
    plsc.subcore_barrier()

    pltpu.sync_copy(s1s.at[pl.ds(s * NSL, NSL)],
                    s1_h.at[pl.ds(c * NP + s * NSL, NSL)])
    pltpu.sync_copy(eav, expa_h.at[pl.ds(eb, EPT)])


@functools.partial(
    pl.kernel,
    out_type=(
        jax.ShapeDtypeStruct((NS, NB3, B), jnp.float32),
        jax.ShapeDtypeStruct((NS, NB3, B), jnp.float32),
        jax.ShapeDtypeStruct((NC * NP,), jnp.float32),
    ),
    mesh=_MESH,
    compiler_params=_SC_PARAMS,
    scratch_types=[
        pltpu.VMEM((N,), jnp.float32),
        pltpu.VMEM((NP,), jnp.float32),
        pltpu.VMEM((NP,), jnp.float32),
        pltpu.VMEM((EPT,), jnp.int32),
        pltpu.VMEM((NB, B), jnp.int32),
        pltpu.VMEM((EPT,), jnp.float32),
        pltpu.VMEM((EPT,), jnp.float32),
        pltpu.VMEM((NB, B), jnp.float32),
        pltpu.VMEM((NB, B), jnp.float32),
        pltpu.VMEM((NSL,), jnp.float32),
        pltpu.VMEM_SHARED((NP,), jnp.float32),
        pltpu.SemaphoreType.DMA,
    ],
)
def _sc_pass2(src_h, dst2d_h, expa_h, u_h, t_h, s1_h, gamma_h, exps_h, s2_h,
              tv, s1v, s1b, sv, dv2d, eav, uv, gv2, esv2, zv, s2s, sem):
    c = lax.axis_index("c")
    s = lax.axis_index("s")
    wid = _worker(c, s)
    eb = wid * EPT

    pltpu.sync_copy(s1_h.at[pl.ds(0, NP)], s1v)
    pltpu.sync_copy(s1_h.at[pl.ds(NP, NP)], s1b)

    def comb(i, _):
        sl = pl.ds(i * 16, 16)
        s1v[sl] = s1v[sl] + s1b[sl]
        return 0

    lax.fori_loop(0, NP // 16, comb, 0)

    pltpu.sync_copy(t_h, tv)
    pltpu.sync_copy(src_h.at[pl.ds(eb, EPT)], sv)
    pltpu.sync_copy(dst2d_h.at[wid], dv2d)
    pltpu.sync_copy(expa_h.at[pl.ds(eb, EPT)], eav)
    pltpu.sync_copy(u_h.at[pl.ds(eb, EPT)], uv)

    for i in range(NSL // 16):
        zv[pl.ds(i * 16, 16)] = _zeros16()
    pltpu.sync_copy(zv, s2s.at[pl.ds(s * NSL, NSL)])
    plsc.subcore_barrier()

    def body(j, _):
        for k in range(B // 16):
            sl = pl.ds(j * B + k * 16, 16)
            sl2 = pl.ds(k * 16, 16)
            s16 = sv[sl]
            g16 = eav[sl] / plsc.load_gather(s1v, [s16])
            gv2[j, sl2] = g16
            sc16 = plsc.load_gather(tv, [s16]) + g16 * uv[sl]
            esv2[j, sl2] = jnp.exp(sc16)
        return 0

    lax.fori_loop(0, NB, body, 0)

    def scat(j, _):
        pltpu.async_copy(esv2.at[j], s2s.at[dv2d.at[j]], sem,
                         add=True)

        @pl.when(j >= 8)
        def _():
            pltpu.make_async_copy(esv2.at[0], s2s.at[dv2d.at[0]],
                                  sem).wait()
        return 0

    lax.fori_loop(0, NB, scat, 0)
    for _ in range(8):
        pltpu.make_async_copy(esv2.at[0], s2s.at[dv2d.at[0]],
                              sem).wait()
    plsc.subcore_barrier()

    pltpu.sync_copy(s2s.at[pl.ds(s * NSL, NSL)],
                    s2_h.at[pl.ds(c * NP + s * NSL, NSL)])
    s3 = c * (NS // 2) + s // 2
    jb = (s % 2) * NB
    pltpu.sync_copy(gv2, gamma_h.at[s3, pl.ds(jb, NB)])
    pltpu.sync_copy(esv2, exps_h.at[s3, pl.ds(jb, NB)])


EPT3 = E // NS
NB3 = EPT3 // B
DH = D // 2
DC = DH + ED
NBS = 50
NSC = NB3 // NBS
NCMB = 2048


def _bcast2d(ref, row, col):
    return plsc.load_gather(ref, [jnp.full((16,), row, jnp.int32),
                                  jnp.full((16,), col, jnp.int32)])


@functools.partial(
    pl.kernel,
    out_type=jax.ShapeDtypeStruct((NC, NP, DC), jnp.float32),
    mesh=_MESH,
    compiler_params=_SC_PARAMS,
    scratch_types=[
        pltpu.VMEM((NP,), jnp.float32),
        pltpu.VMEM((NCMB,), jnp.float32),
        pltpu.VMEM((NBS, B), jnp.int32),
        pltpu.VMEM((NBS, B), jnp.int32),
        pltpu.VMEM((NBS, B), jnp.float32),
        pltpu.VMEM((NBS, B), jnp.float32),
        pltpu.VMEM((B, DH), jnp.float32),
        pltpu.VMEM((B, DH), jnp.float32),
        pltpu.VMEM((ED, B), jnp.float32),
        pltpu.VMEM((ED, B), jnp.float32),
        pltpu.VMEM((B, DC), jnp.float32),
        pltpu.VMEM((B, DC), jnp.float32),
        pltpu.SemaphoreType.DMA,
        pltpu.SemaphoreType.DMA,
        pltpu.SemaphoreType.DMA,
        pltpu.SemaphoreType.DMA,
        pltpu.SemaphoreType.DMA,
        pltpu.SemaphoreType.DMA,
        pltpu.VMEM_SHARED((NP, DC), jnp.float32),
    ],
)
def _sc_pass3(src3_h, dst3_h, exps3_h, gamma3_h, s2_h, x0_h, x1_h, eft_h,
              acc_h,
              s2v, ctmp, svx, dvx, esx, gvx, gbuf0, gbuf1, efg0, efg1,
              sbuf0, sbuf1, sem_g0, sem_g1, sem_e0, sem_e1, sem_s0,
              sem_s1, accs):
    c = lax.axis_index("c")
    s = lax.axis_index("s")
    eb = s * EPT3
    gbufs = (gbuf0, gbuf1)
    efgs = (efg0, efg1)
    sbufs = (sbuf0, sbuf1)
    sem_gs = (sem_g0, sem_g1)
    sem_es = (sem_e0, sem_e1)
    sem_ss = (sem_s0, sem_s1)

    def comb(m, _):
        sl = pl.ds(m * NCMB, NCMB)
        pltpu.sync_copy(s2_h.at[sl], s2v.at[sl])
        pltpu.sync_copy(s2_h.at[pl.ds(NP + m * NCMB, NCMB)], ctmp)

        def add16(i, _):
            sl16 = pl.ds(m * NCMB + i * 16, 16)
            s2v[sl16] = s2v[sl16] + ctmp[pl.ds(i * 16, 16)]
            return 0

        lax.fori_loop(0, NCMB // 16, add16, 0)
        return 0

    lax.fori_loop(0, NP // NCMB, comb, 0)

    def zrow(e, _):
        for k in range(DC // 16):
            sbuf0[e, pl.ds(k * 16, 16)] = _zeros16()
        return 0

    lax.fori_loop(0, B, zrow, 0)
    for k in range(NSL // B):
        pltpu.sync_copy(sbuf0, accs.at[pl.ds(s * NSL + k * B, B)])
    plsc.subcore_barrier()

    def issue_in(j, gj, p):
        @pl.when(c == 0)
        def _():
            pltpu.async_copy(x0_h.at[svx.at[j]], gbufs[p], sem_gs[p])

        @pl.when(c == 1)
        def _():
            pltpu.async_copy(x1_h.at[svx.at[j]], gbufs[p], sem_gs[p])

        pltpu.async_copy(eft_h.at[:, pl.ds(eb + gj * B, B)], efgs[p],
                         sem_es[p])

    def wait_x(p):
        pltpu.make_async_copy(x0_h.at[svx.at[0]], gbufs[p], sem_gs[p]).wait()

    def wait_ef(p):
        pltpu.make_async_copy(eft_h.at[:, pl.ds(eb, B)], efgs[p],
                              sem_es[p]).wait()

    def wait_out(p):
        pltpu.make_async_copy(sbufs[p], accs.at[dvx.at[0]], sem_ss[p]).wait()

    def superchunk(sc, _):
        rsl = pl.ds(sc * NBS, NBS)
        pltpu.sync_copy(src3_h.at[s, rsl], svx)
        pltpu.sync_copy(dst3_h.at[s, rsl], dvx)
        pltpu.sync_copy(exps3_h.at[s, rsl], esx)
        pltpu.sync_copy(gamma3_h.at[s, rsl], gvx)

        def arow(j, _):
            for k in range(B // 16):
                sl = pl.ds(k * 16, 16)
                a16 = esx[j, sl] / plsc.load_gather(s2v, [dvx[j, sl]])
                esx[j, sl] = a16
                gvx[j, sl] = (a16 * 0.5) * gvx[j, sl]
            return 0

        lax.fori_loop(0, NBS, arow, 0)

        csc = sc * NBS

        iota = lax.iota(jnp.int32, 16)

        def slot(j, p):
            @pl.when(j + 1 < NBS)
            def _():
                issue_in(j + 1, csc + j + 1, 1 - p)

            @pl.when(j >= 2)
            def _():
                wait_out(p)

            wait_ef(p)
            jcol = jnp.full((16,), j, jnp.int32)

            def scale_e(m, _):
                sl = pl.ds(m * 16, 16)
                v16 = gvx[j, sl]
                rows16 = iota + m * 16
                for k in range(ED):
                    plsc.store_scatter(
                        sbufs[p], [rows16, jnp.full((16,), DH + k, jnp.int32)],
                        efgs[p][k, sl] * v16)
                return 0

            lax.fori_loop(0, B // 16, scale_e, 0)

            wait_x(p)

            def scale(e2, _):
                for e in (2 * e2, 2 * e2 + 1):
                    ab = plsc.load_gather(
                        esx, [jcol, jnp.full((16,), e, jnp.int32)])
                    for k in range(DH // 16):
                        sl = pl.ds(k * 16, 16)
                        sbufs[p][e, sl] = gbufs[p][e, sl] * ab
                return 0

            lax.fori_loop(0, B // 2, scale, 0)
            pltpu.async_copy(sbufs[p], accs.at[dvx.at[j]], sem_ss[p],
                             add=True)
            return 0

        def pair(jj, _):
            slot(2 * jj, 0)
            slot(2 * jj + 1, 1)
            return 0

        issue_in(0, csc, 0)
        lax.fori_loop(0, NBS // 2, pair, 0)
        wait_out(0)
        wait_out(1)
        return 0

    lax.fori_loop(0, NSC, superchunk, 0)
    plsc.subcore_barrier()

    pltpu.sync_copy(accs.at[pl.ds(s * NSL, NSL)],
                    acc_h.at[c, pl.ds(s * NSL, NSL)])


def _proj_body(x_ref, w_ref, o_ref):
    o_ref[...] = jnp.dot(x_ref[...], w_ref[...],
                         preferred_element_type=jnp.float32)


def _node_proj(x, w3):
    blk = 1000
    return pl.pallas_call(
        _proj_body,
        grid=(N // blk,),
        in_specs=[
            pl.BlockSpec((blk, D), lambda i: (i, 0)),
            pl.BlockSpec((D, 8), lambda i: (0, 0)),
        ],
        out_specs=pl.BlockSpec((blk, 8), lambda i: (i, 0)),
        out_shape=jax.ShapeDtypeStruct((N, 8), jnp.float32),
    )(x, w3)


def _edge_proj_body(eft_ref, we_ref, w2_ref, r_ref, u_ref):
    blk = eft_ref[...]
    r_ref[...] = jnp.sum(blk * we_ref[...], axis=0)
    u_ref[...] = jnp.sum(blk * w2_ref[...], axis=0)


def _edge_proj(eft, we_col, w2_col):
    return pl.pallas_call(
        _edge_proj_body,
        grid=(1,),
        in_specs=[
            pl.BlockSpec((ED, E), lambda i: (0, 0)),
            pl.BlockSpec((ED, 1), lambda i: (0, 0)),
            pl.BlockSpec((ED, 1), lambda i: (0, 0)),
        ],
        out_specs=[
            pl.BlockSpec((E,), lambda i: (0,)),
            pl.BlockSpec((E,), lambda i: (0,)),
        ],
        out_shape=[
            jax.ShapeDtypeStruct((E,), jnp.float32),
            jax.ShapeDtypeStruct((E,), jnp.float32),
        ],
    )(eft, we_col, w2_col)


def _final_body(x_ref, a0_ref, a1_ref, wn_ref, wc0_ref, wc1_ref, o_ref):
    acc = jnp.dot(x_ref[...], wn_ref[...], preferred_element_type=jnp.float32)
    acc = acc + jnp.dot(a0_ref[...], wc0_ref[...],
                        preferred_element_type=jnp.float32)
    acc = acc + jnp.dot(a1_ref[...], wc1_ref[...],
                        preferred_element_type=jnp.float32)
    o_ref[...] = acc


def _final_linear(x, a0, a1, wn_t, wc0, wc1):
    blk = 1000
    return pl.pallas_call(
        _final_body,
        grid=(N // blk,),
        in_specs=[
            pl.BlockSpec((blk, D), lambda i: (i, 0)),
            pl.BlockSpec((blk, DC), lambda i: (i, 0)),
            pl.BlockSpec((blk, DC), lambda i: (i, 0)),
            pl.BlockSpec((D, D), lambda i: (0, 0)),
            pl.BlockSpec((DC, D), lambda i: (0, 0)),
            pl.BlockSpec((DC, D), lambda i: (0, 0)),
        ],
        out_specs=pl.BlockSpec((blk, D), lambda i: (i, 0)),
        out_shape=jax.ShapeDtypeStruct((N, D), jnp.float32),
    )(x, a0, a1, wn_t, wc0, wc1)


def kernel(node_features, edge_features, edge_index, W_group_attn,
           W_src_attn, W_linear):
    x = node_features
    eft = edge_features.T
    src = edge_index[0]
    dst = edge_index[1]
    src2d, dst2d, src2d3, dst2d3 = lax.optimization_barrier(
        (src.reshape(NW, NB, B), dst.reshape(NW, NB, B),
         src.reshape(NS, NB3, B), dst.reshape(NS, NB3, B)))
    x0 = x[:, :DH]
    x1 = x[:, DH:]

    w3 = jnp.zeros((D, 8), jnp.float32)
    w3 = w3.at[:, 0].set(W_group_attn[0, ED:ED + D])
    w3 = w3.at[:, 1].set(W_group_attn[0, ED + D:ED + 2 * D])
    w3 = w3.at[:, 2].set(W_src_attn[0, :D])
    we_col = W_group_attn[0, :ED].reshape(ED, 1)
    w2_col = W_src_attn[0, D:D + ED].reshape(ED, 1)

    pqt = _node_proj(x, w3)
    r, u = _edge_proj(eft, we_col, w2_col)
    p = pqt[:, 0]
    q = pqt[:, 1]
    t = pqt[:, 2]

    expa, s1 = _sc_pass1(src, src2d, dst, r, p, q)
    gamma3, exps3, s2 = _sc_pass2(src, dst2d, expa, u, t, s1)
    acc = _sc_pass3(src2d3, dst2d3, exps3, gamma3, s2, x0, x1, eft)

    wn_t = W_linear[:, :D].T
    wc0 = jnp.concatenate([W_linear[:, D:D + DH].T,
                           W_linear[:, 2 * D:2 * D + ED].T], axis=0)
    wc1 = jnp.concatenate([W_linear[:, D + DH:2 * D].T,
                           W_linear[:, 2 * D:2 * D + ED].T], axis=0)
    return _final_linear(x, acc[0, :N], acc[1, :N], wn_t, wc0, wc1)
